# Initial kernel scaffold; baseline (speedup 1.0000x reference)
import jax
import jax.numpy as jnp
from jax.experimental import pallas as pl


def kernel(x, kernels, channel_combinations, biases):
    raise NotImplementedError("write your pallas kernel here")


# speedup vs baseline: 4.0627x; 4.0627x over previous
import itertools

import jax
import jax.numpy as jnp
import numpy as np
from jax.experimental import pallas as pl
from jax.experimental.pallas import tpu as pltpu

C_IN, SEQ_LEN = 9, 2048
KERNEL_SIZE = 9
NUM_KERNELS = 84
NUM_FEATURES = 10000
MAX_DIL_PER_KERNEL = 32
BATCH = 32


def _cfg():
    nf_total = NUM_FEATURES // 2
    nf_total = nf_total // NUM_KERNELS * NUM_KERNELS
    nfpk = nf_total // NUM_KERNELS
    true_max = min(nfpk, MAX_DIL_PER_KERNEL)
    multiplier = nfpk / true_max
    max_exp = np.log2((SEQ_LEN - 1) / (KERNEL_SIZE - 1))
    dilations, counts = np.unique(
        np.logspace(0, max_exp, true_max, base=2).astype(np.int32),
        return_counts=True)
    nfpd = (counts * multiplier).astype(np.int32)
    rem = nfpk - nfpd.sum()
    i = 0
    while rem > 0:
        nfpd[i] += 1
        rem -= 1
        i = (i + 1) % len(nfpd)
    return [int(d) for d in dilations], [int(n) for n in nfpd]


_DILS, _NFPD = _cfg()
_D = len(_DILS)
_NF_SUM = sum(_NFPD)
_HALF = NUM_KERNELS // 2
_SLOTS = 2 * _NF_SUM
_OUT_W = 4 * _HALF * _NF_SUM

_PERM = list(range(0, NUM_KERNELS, 2)) + list(range(1, NUM_KERNELS, 2))


def _final_perm():
    pidx = np.zeros(_OUT_W, np.int32)
    base = 0
    boff = 0
    for i, nf in enumerate(_NFPD):
        p1 = i % 2
        c0 = 2 * boff
        rA = 0 if p1 == 0 else _HALF
        rB = _HALF - rA
        for seg, (r0, cadd) in enumerate(
                [(rA, 0), (rA, nf), (rB, 0), (rB, nf)]):
            for j in range(_HALF):
                for f in range(nf):
                    pidx[base + seg * _HALF * nf + j * nf + f] = (
                        (r0 + j) * _SLOTS + (c0 + cadd + f))
        base += 4 * _HALF * nf
        boff += nf
    return pidx


_PIDX = jnp.asarray(_final_perm())


def _body(x_ref, w_ref, b_ref, o_ref):
    L = SEQ_LEN
    x = x_ref[0]
    bias_all = b_ref[...]
    cols = []
    boff = 0
    for i, (d, nf) in enumerate(zip(_DILS, _NFPD)):
        p = (KERNEL_SIZE - 1) * d // 2
        shifts = []
        for t in range(KERNEL_SIZE):
            o = (t - KERNEL_SIZE // 2) * d
            if o == 0:
                shifts.append(x)
            elif o > 0:
                z = jnp.zeros((C_IN, o), jnp.float32)
                shifts.append(jnp.concatenate([x[:, o:], z], axis=1))
            else:
                z = jnp.zeros((C_IN, -o), jnp.float32)
                shifts.append(jnp.concatenate([z, x[:, :o]], axis=1))
        xs = jnp.concatenate(shifts, axis=0)
        C = jnp.dot(w_ref[i], xs,
                    preferred_element_type=jnp.float32)

        p1 = i % 2
        rF = 0 if p1 == 0 else _HALF
        rC = _HALF - rF
        Cf = C[rF:rF + _HALF]
        Cc = C[rC:rC + _HALF, p:L - p]
        Lc = L - 2 * p
        sf = jnp.sum(Cf, axis=1, keepdims=True)
        sc = jnp.sum(Cc, axis=1, keepdims=True)
        a_cols = []
        b_cols = []
        for f in range(nf):
            bcol = bias_all[:, boff + f:boff + f + 1]
            bF = bcol[rF:rF + _HALF]
            bC = bcol[rC:rC + _HALF]

            difF = Cf - bF
            cntF = jnp.sum((difF > 0).astype(jnp.float32), axis=1,
                           keepdims=True)
            absF = jnp.sum(jnp.abs(difF), axis=1, keepdims=True)
            reluF = 0.5 * ((sf - L * bF) + absF)
            aF = cntF * (1.0 / L)
            bfF = reluF / jnp.maximum(absF, 1e-8)

            difC = Cc - bC
            cntC = jnp.sum((difC > 0).astype(jnp.float32), axis=1,
                           keepdims=True)
            absC = jnp.sum(jnp.abs(difC), axis=1, keepdims=True)
            reluC = 0.5 * ((sc - Lc * bC) + absC)
            aC = cntC * (1.0 / Lc)
            bfC = reluC / jnp.maximum(absC, 1e-8)

            if p1 == 0:
                a_cols.append(jnp.concatenate([aF, aC], axis=0))
                b_cols.append(jnp.concatenate([bfF, bfC], axis=0))
            else:
                a_cols.append(jnp.concatenate([aC, aF], axis=0))
                b_cols.append(jnp.concatenate([bfC, bfF], axis=0))
        cols.extend(a_cols)
        cols.extend(b_cols)
        boff += nf
    o_ref[0] = jnp.concatenate(cols, axis=1)


def kernel(x, kernels, channel_combinations, biases):
    B = x.shape[0]
    kw = kernels[:NUM_KERNELS, 0, :]
    cc = channel_combinations.transpose(0, 2, 1)
    W = kw[None, :, :, None] * cc[:, :, None, :]
    W = W.reshape(_D, NUM_KERNELS, KERNEL_SIZE * C_IN)
    W = W[:, _PERM, :]
    b_perm = biases[jnp.asarray(_PERM, jnp.int32), :]

    staged = pl.pallas_call(
        _body,
        out_shape=jax.ShapeDtypeStruct((B, NUM_KERNELS, _SLOTS), jnp.float32),
        grid=(B,),
        in_specs=[
            pl.BlockSpec((1, C_IN, SEQ_LEN), lambda i: (i, 0, 0)),
            pl.BlockSpec((_D, NUM_KERNELS, KERNEL_SIZE * C_IN),
                         lambda i: (0, 0, 0)),
            pl.BlockSpec((NUM_KERNELS, _NF_SUM), lambda i: (0, 0)),
        ],
        out_specs=pl.BlockSpec((1, NUM_KERNELS, _SLOTS), lambda i: (i, 0, 0)),
        compiler_params=pltpu.CompilerParams(
            dimension_semantics=("parallel",),
        ),
        name="minirocket_features",
    )(x, W, b_perm)

    return staged.reshape(B, NUM_KERNELS * _SLOTS)[:, _PIDX]
